# baseline (speedup 1.0000x reference)
import functools
import math

import jax
import jax.numpy as jnp
from jax import lax
from jax.experimental import pallas as pl
from jax.experimental.pallas import tpu as pltpu
from jax.experimental.pallas import tpu_sc as plsc

D_MODEL = 128
SCALE = math.sqrt(float(D_MODEL))
NUM_CORES = 2
NUM_SUBCORES = 16
NUM_WORKERS = NUM_CORES * NUM_SUBCORES
LANES = 16
CHUNK = 64
NBUF = 5


def _make_sc_kernel(n_chunks: int, total_rows: int):
    assert n_chunks % NBUF == 0
    per_worker = n_chunks * CHUNK
    mesh = plsc.VectorSubcoreMesh(
        core_axis_name="c", subcore_axis_name="s",
        num_cores=NUM_CORES, num_subcores=NUM_SUBCORES)

    @functools.partial(
        pl.kernel,
        out_type=jax.ShapeDtypeStruct((total_rows, D_MODEL), jnp.float32),
        mesh=mesh,
        scratch_types=[
            pltpu.VMEM((n_chunks, CHUNK), jnp.int32),
            pltpu.VMEM((NBUF, CHUNK, D_MODEL), jnp.float32),
            pltpu.VMEM_SHARED(
                (NUM_SUBCORES, NBUF, CHUNK, D_MODEL), jnp.float32),
            pltpu.SemaphoreType.DMA((NBUF,)),
            pltpu.SemaphoreType.DMA((NBUF,)),
            pltpu.SemaphoreType.DMA((NBUF,)),
        ],
    )
    def sc_kernel(idx_hbm, table_hbm, out_hbm, idx_v, bufs, stage,
                  gsem, xsem, hsem):
        wid = lax.axis_index("s") * NUM_CORES + lax.axis_index("c")
        sid = lax.axis_index("s")
        base = wid * per_worker
        pltpu.sync_copy(idx_hbm.at[wid], idx_v)

        def gather(j, b):
            return pltpu.make_async_copy(
                table_hbm.at[idx_v.at[j]], bufs.at[b], gsem.at[b])

        def up(b, r):
            return pltpu.make_async_copy(
                bufs.at[b], stage.at[sid, r], xsem.at[r])

        def down(j, r):
            return pltpu.make_async_copy(
                stage.at[sid, r], out_hbm.at[pl.ds(base + j * CHUNK, CHUNK)],
                hsem.at[r])

        for b in range(NBUF - 1):
            gather(b, b).start()

        def outer(g, carry):
            j0 = g * NBUF
            for t in range(NBUF):
                j = j0 + t
                rp = (t + NBUF - 1) % NBUF

                @pl.when(j >= 1)
                def _():
                    up(rp, rp).wait()
                    down(j - 1, rp).start()

                jn = j + NBUF - 1

                @pl.when(jn < n_chunks)
                def _():
                    gather(jn, rp).start()

                gather(j, t).wait()

                @plsc.parallel_loop(0, CHUNK, step=1, unroll=4)
                def _(i):
                    for l in range(D_MODEL // LANES):
                        s = pl.ds(l * LANES, LANES)
                        bufs[t, i, s] = bufs[t, i, s] * SCALE

                @pl.when(j >= NBUF)
                def _():
                    down(j - NBUF, t).wait()
                up(t, t).start()
            return carry

        lax.fori_loop(0, n_chunks // NBUF, outer, 0)

        last = n_chunks - 1
        r_last = last % NBUF
        up(r_last, r_last).wait()
        down(last, r_last).start()
        for b in range(NBUF):
            down(n_chunks - NBUF + b, b).wait()

    return sc_kernel


def kernel(x, table):
    rows, cols = x.shape
    total = rows * cols
    n_chunks = total // (NUM_WORKERS * CHUNK)
    idx = x.reshape(NUM_WORKERS, n_chunks, CHUNK).astype(jnp.int32)
    out = _make_sc_kernel(n_chunks, total)(idx, table)
    return out.reshape(rows, cols, D_MODEL)
